# baseline (speedup 1.0000x reference)
import functools

import jax
import jax.numpy as jnp
from jax.experimental import pallas as pl
from jax.experimental.pallas import tpu as pltpu

H = 4
D = 256
ROW_BLOCK = 2000


def _proj_normalize(xb, wT, b):
    k = jnp.dot(xb, wT, preferred_element_type=jnp.float32)
    k = k + b[None, :]
    outs = []
    for h in range(H):
        kh = k[:, h * D:(h + 1) * D]
        kh = kh * jax.lax.rsqrt(jnp.sum(kh * kh, axis=1, keepdims=True))
        outs.append(kh)
    return outs


def _fused(n_total, x_ref, wT_ref, b_ref, out_ref, kvs_ref, sums_ref):
    p = pl.program_id(0)
    j = pl.program_id(1)
    xb = x_ref[...].astype(jnp.bfloat16)

    @pl.when(p == 0)
    def _phase_a():
        @pl.when(j == 0)
        def _init():
            kvs_ref[...] = jnp.zeros_like(kvs_ref)
            sums_ref[...] = jnp.zeros_like(sums_ref)

        ks = _proj_normalize(xb, wT_ref[0], b_ref[0, 0])
        khat = jnp.concatenate([kh.astype(jnp.bfloat16) for kh in ks], axis=1)
        kvs_ref[...] += jax.lax.dot_general(
            khat, xb, (((0,), (0,)), ((), ())),
            preferred_element_type=jnp.float32)
        rows = [jnp.sum(kh, axis=0)[None, :] for kh in ks]
        rows.append(jnp.sum(x_ref[...], axis=0)[None, :])
        rows.append(jnp.zeros((3, D), jnp.float32))
        sums_ref[...] += jnp.concatenate(rows, axis=0)

    @pl.when(p == 1)
    def _phase_b():
        x_sum = sums_ref[H, :]
        qs = _proj_normalize(xb, wT_ref[0], b_ref[0, 0])
        kvs_b = kvs_ref[...].astype(jnp.bfloat16)
        for h in range(H):
            q = qs[h]
            num = jnp.dot(q.astype(jnp.bfloat16),
                          kvs_b[h * D:(h + 1) * D, :],
                          preferred_element_type=jnp.float32)
            num = num + x_sum[None, :]
            den = jnp.sum(q * sums_ref[h, :][None, :], axis=1, keepdims=True)
            den = den + jnp.float32(n_total)
            out_ref[:, h * D:(h + 1) * D] = num / den


def kernel(x, Wq, bq, Wk, bk):
    n, in_ch = x.shape
    assert n % ROW_BLOCK == 0
    nb = n // ROW_BLOCK
    wT = jnp.stack([Wk.T.astype(jnp.bfloat16), Wq.T.astype(jnp.bfloat16)])
    b2 = jnp.stack([bk[None, :], bq[None, :]])

    out = pl.pallas_call(
        functools.partial(_fused, n),
        grid=(2, nb),
        in_specs=[
            pl.BlockSpec((ROW_BLOCK, in_ch), lambda p, j: (j, 0)),
            pl.BlockSpec((1, in_ch, H * D), lambda p, j: (p, 0, 0)),
            pl.BlockSpec((1, 1, H * D), lambda p, j: (p, 0, 0)),
        ],
        out_specs=pl.BlockSpec((ROW_BLOCK, H * D), lambda p, j: (p * j, 0)),
        out_shape=jax.ShapeDtypeStruct((n, H * D), jnp.float32),
        scratch_shapes=[
            pltpu.VMEM((H * D, D), jnp.float32),
            pltpu.VMEM((8, D), jnp.float32),
        ],
    )(x, wT, b2)
    return out


# speedup vs baseline: 1.2063x; 1.2063x over previous
import functools

import jax
import jax.numpy as jnp
from jax.experimental import pallas as pl
from jax.experimental.pallas import tpu as pltpu

H = 4
D = 256
ROW_BLOCK = 5000


def _fused(n_total, x_ref, wT_ref, b_ref, out_ref, kvs_ref, sums_ref):
    p = pl.program_id(0)
    j = pl.program_id(1)
    xb = x_ref[...].astype(jnp.bfloat16)

    @pl.when(p == 0)
    def _phase_a():
        @pl.when(j == 0)
        def _init():
            kvs_ref[...] = jnp.zeros_like(kvs_ref)
            sums_ref[...] = jnp.zeros_like(sums_ref)

        rows = []
        for h in range(H):
            k = jnp.dot(xb, wT_ref[0, :, h * D:(h + 1) * D],
                        preferred_element_type=jnp.float32)
            k = k + b_ref[0, 0, h * D:(h + 1) * D][None, :]
            k = k * jax.lax.rsqrt(jnp.sum(k * k, axis=1, keepdims=True))
            kvs_ref[h] += jax.lax.dot_general(
                k.astype(jnp.bfloat16), xb, (((0,), (0,)), ((), ())),
                preferred_element_type=jnp.float32)
            rows.append(jnp.sum(k, axis=0)[None, :])
        rows.append(jnp.sum(x_ref[...], axis=0)[None, :])
        rows.append(jnp.zeros((3, D), jnp.float32))
        sums_ref[...] += jnp.concatenate(rows, axis=0)

    @pl.when(p == 1)
    def _phase_b():
        x_sum = sums_ref[H, :]
        for h in range(H):
            q = jnp.dot(xb, wT_ref[0, :, h * D:(h + 1) * D],
                        preferred_element_type=jnp.float32)
            q = q + b_ref[0, 0, h * D:(h + 1) * D][None, :]
            q = q * jax.lax.rsqrt(jnp.sum(q * q, axis=1, keepdims=True))
            num = jnp.dot(q.astype(jnp.bfloat16),
                          kvs_ref[h].astype(jnp.bfloat16),
                          preferred_element_type=jnp.float32)
            num = num + x_sum[None, :]
            den = jnp.sum(q * sums_ref[h, :][None, :], axis=1, keepdims=True)
            den = den + jnp.float32(n_total)
            out_ref[:, h * D:(h + 1) * D] = num / den


def kernel(x, Wq, bq, Wk, bk):
    n, in_ch = x.shape
    assert n % ROW_BLOCK == 0
    nb = n // ROW_BLOCK
    wT = jnp.stack([Wk.T.astype(jnp.bfloat16), Wq.T.astype(jnp.bfloat16)])
    b2 = jnp.stack([bk[None, :], bq[None, :]])

    out = pl.pallas_call(
        functools.partial(_fused, n),
        grid=(2, nb),
        in_specs=[
            pl.BlockSpec((ROW_BLOCK, in_ch), lambda p, j: (j, 0)),
            pl.BlockSpec((1, in_ch, H * D), lambda p, j: (p, 0, 0)),
            pl.BlockSpec((1, 1, H * D), lambda p, j: (p, 0, 0)),
        ],
        out_specs=pl.BlockSpec((ROW_BLOCK, H * D), lambda p, j: (p * j, 0)),
        out_shape=jax.ShapeDtypeStruct((n, H * D), jnp.float32),
        scratch_shapes=[
            pltpu.VMEM((H, D, D), jnp.float32),
            pltpu.VMEM((8, D), jnp.float32),
        ],
    )(x, wT, b2)
    return out


# speedup vs baseline: 1.2708x; 1.0535x over previous
import functools

import jax
import jax.numpy as jnp
from jax.experimental import pallas as pl
from jax.experimental.pallas import tpu as pltpu

H = 4
D = 256
BLOCK_A = 5000
BLOCK_B = 2000


def _fused(n_total, nb_a, xa_ref, xb_ref, wT_ref, b_ref, out_ref,
           kvs_ref, sums_ref):
    s = pl.program_id(0)

    @pl.when(s < nb_a)
    def _phase_a():
        @pl.when(s == 0)
        def _init():
            kvs_ref[...] = jnp.zeros_like(kvs_ref)
            sums_ref[...] = jnp.zeros_like(sums_ref)

        xb = xa_ref[...].astype(jnp.bfloat16)
        rows = []
        for h in range(H):
            k = jnp.dot(xb, wT_ref[0, :, h * D:(h + 1) * D],
                        preferred_element_type=jnp.float32)
            k = k + b_ref[0, 0, h * D:(h + 1) * D][None, :]
            k = k * jax.lax.rsqrt(jnp.sum(k * k, axis=1, keepdims=True))
            kvs_ref[h] += jax.lax.dot_general(
                k.astype(jnp.bfloat16), xb, (((0,), (0,)), ((), ())),
                preferred_element_type=jnp.float32)
            rows.append(jnp.sum(k, axis=0)[None, :])
        rows.append(jnp.sum(xa_ref[...], axis=0)[None, :])
        rows.append(jnp.zeros((3, D), jnp.float32))
        sums_ref[...] += jnp.concatenate(rows, axis=0)

    @pl.when(s >= nb_a)
    def _phase_b():
        xb = xb_ref[...].astype(jnp.bfloat16)
        x_sum = sums_ref[H, :]
        for h in range(H):
            q = jnp.dot(xb, wT_ref[0, :, h * D:(h + 1) * D],
                        preferred_element_type=jnp.float32)
            q = q + b_ref[0, 0, h * D:(h + 1) * D][None, :]
            q = q * jax.lax.rsqrt(jnp.sum(q * q, axis=1, keepdims=True))
            num = jnp.dot(q.astype(jnp.bfloat16),
                          kvs_ref[h].astype(jnp.bfloat16),
                          preferred_element_type=jnp.float32)
            num = num + x_sum[None, :]
            den = jnp.sum(q * sums_ref[h, :][None, :], axis=1, keepdims=True)
            den = den + jnp.float32(n_total)
            out_ref[:, h * D:(h + 1) * D] = num / den


def kernel(x, Wq, bq, Wk, bk):
    n, in_ch = x.shape
    assert n % BLOCK_A == 0 and n % BLOCK_B == 0
    nb_a = n // BLOCK_A
    nb_b = n // BLOCK_B
    wT = jnp.stack([Wk.T.astype(jnp.bfloat16), Wq.T.astype(jnp.bfloat16)])
    b2 = jnp.stack([bk[None, :], bq[None, :]])

    out = pl.pallas_call(
        functools.partial(_fused, n, nb_a),
        grid=(nb_a + nb_b,),
        in_specs=[
            pl.BlockSpec((BLOCK_A, in_ch),
                         lambda s: (jnp.minimum(s, nb_a - 1), 0)),
            pl.BlockSpec((BLOCK_B, in_ch),
                         lambda s: (jnp.maximum(s - nb_a, 0), 0)),
            pl.BlockSpec((1, in_ch, H * D),
                         lambda s: (jnp.where(s < nb_a, 0, 1), 0, 0)),
            pl.BlockSpec((1, 1, H * D),
                         lambda s: (jnp.where(s < nb_a, 0, 1), 0, 0)),
        ],
        out_specs=pl.BlockSpec((BLOCK_B, H * D),
                               lambda s: (jnp.maximum(s - nb_a, 0), 0)),
        out_shape=jax.ShapeDtypeStruct((n, H * D), jnp.float32),
        scratch_shapes=[
            pltpu.VMEM((H, D, D), jnp.float32),
            pltpu.VMEM((8, D), jnp.float32),
        ],
    )(x, x, wT, b2)
    return out
